# baseline (speedup 1.0000x reference)
import functools

import jax
import jax.numpy as jnp
from jax import lax
from jax.experimental import pallas as pl
from jax.experimental.pallas import tpu as pltpu
from jax.experimental.pallas import tpu_sc as plsc

N = 10000
E = 320000
D = 128

NC = 2
NS = 16
NW = NC * NS

N_PAD = 10240
HALF_N = N_PAD // NC
QN = N_PAD // 4
ACC_ROWS = QN + 8
QROWS_PT = QN // NS
CH = 128
NROW = 2560
E_PAD = NROW * CH
NCH_DEG = NROW // NW
NCH_EDGE = NROW // NS
NPASS = 2
PCH = NCH_EDGE // NPASS
GRP = 4
EGRP = 4
GEDGES = EGRP * CH
CAP = PCH * CH + GEDGES + 256

BLK = 1280
GRID = N_PAD // BLK

_mesh = plsc.VectorSubcoreMesh(core_axis_name="c", subcore_axis_name="s")
_sc_params = pltpu.CompilerParams(use_tc_tiling_on_sc=False,
                                  needs_layout_passes=False)


@functools.partial(
    pl.kernel,
    out_type=jax.ShapeDtypeStruct((NW, N_PAD, 8), jnp.float32),
    mesh=_mesh,
    scratch_types=[
        pltpu.VMEM((NCH_DEG, CH), jnp.int32),
        pltpu.VMEM((N_PAD, 8), jnp.float32),
    ],
    compiler_params=_sc_params,
)
def _deg_kernel(dst_hbm, zeros_hbm, out_hbm, dst_v, hist):
    c = lax.axis_index("c")
    t = lax.axis_index("s")
    w = t * NC + c

    pltpu.sync_copy(dst_hbm.at[pl.ds(w * NCH_DEG, NCH_DEG)], dst_v)
    pltpu.sync_copy(zeros_hbm, hist)

    lanes = lax.iota(jnp.int32, 16)
    cols = lanes & 7
    mlow = lanes < 8
    mhigh = lanes >= 8
    ones_v = jnp.ones((16,), jnp.float32)

    def body(r, carry):
        for q in range(CH // 16):
            d_ = dst_v[r, pl.ds(16 * q, 16)]
            plsc.addupdate_scatter(hist, [d_, cols], ones_v, mask=mlow)
            plsc.addupdate_scatter(hist, [d_, cols], ones_v, mask=mhigh)
        return carry

    lax.fori_loop(0, NCH_DEG, body, 0)

    pltpu.sync_copy(hist, out_hbm.at[w])


@functools.partial(
    pl.kernel,
    out_type=jax.ShapeDtypeStruct((N_PAD, D), jnp.float32),
    mesh=_mesh,
    scratch_types=[
        pltpu.VMEM((PCH, CH), jnp.int32),
        pltpu.VMEM((PCH, CH), jnp.int32),
        pltpu.VMEM((PCH, CH), jnp.int32),
        pltpu.VMEM((PCH, CH), jnp.int32),
        pltpu.VMEM((EGRP, CH, D), jnp.float32),
        pltpu.VMEM_SHARED((ACC_ROWS, D), jnp.float32),
        pltpu.SemaphoreType.DMA,
        pltpu.SemaphoreType.DMA,
    ],
    compiler_params=_sc_params,
)
def _edge_kernel(y_hbm, src_hbm, dst_hbm, dmy_s_hbm, dmy_d_hbm, out_hbm,
                 src_v, dst_v, csrc, cdst, rows_v, acc, sem, sem_s):
    c = lax.axis_index("c")
    t = lax.axis_index("s")
    asl = pl.ds(t * QROWS_PT, QROWS_PT)

    for phase in range(2):
        lo = (phase * NC + c) * QN
        osl = pl.ds(lo + t * QROWS_PT, QROWS_PT)

        pltpu.sync_copy(y_hbm.at[osl], acc.at[asl])
        plsc.subcore_barrier()

        for p in range(NPASS):
            row0 = t * NCH_EDGE + p * PCH
            pltpu.sync_copy(src_hbm.at[pl.ds(row0, PCH)], src_v)
            pltpu.sync_copy(dst_hbm.at[pl.ds(row0, PCH)], dst_v)
            pltpu.sync_copy(dmy_s_hbm, csrc)
            pltpu.sync_copy(dmy_d_hbm, cdst)

            def crow(r, cnt_vec):
                for q in range(CH // 16):
                    s = src_v[r, pl.ds(16 * q, 16)]
                    d_ = dst_v[r, pl.ds(16 * q, 16)]
                    m = (d_ >= lo) & (d_ < lo + QN)
                    dl = d_ - lo
                    m_i = jnp.where(m, 1, 0)
                    pos = plsc.cumsum(m_i) - m_i + cnt_vec
                    prow = lax.shift_right_logical(pos, 7)
                    pcol = lax.bitwise_and(pos, 127)
                    plsc.store_scatter(csrc, [prow, pcol], s, mask=m)
                    plsc.store_scatter(cdst, [prow, pcol], dl, mask=m)
                    cnt_vec = cnt_vec + plsc.all_reduce_population_count(m)
                return cnt_vec

            cnt_vec = lax.fori_loop(0, PCH, crow, jnp.zeros((16,), jnp.int32))
            cnt = jnp.sum(cnt_vec) // 16

            def sbody(g, carry):
                gathers = [
                    pltpu.async_copy(y_hbm.at[csrc.at[g * EGRP + k]],
                                     rows_v.at[k], sem)
                    for k in range(EGRP)
                ]
                for g_ in gathers:
                    g_.wait()
                scatters = [
                    pltpu.async_copy(rows_v.at[k],
                                     acc.at[cdst.at[g * EGRP + k]],
                                     sem_s, add=True)
                    for k in range(EGRP)
                ]
                for s_ in scatters:
                    s_.wait()
                return carry

            lax.fori_loop(0, PCH // EGRP, sbody, 0)

        plsc.subcore_barrier()
        pltpu.sync_copy(acc.at[asl], out_hbm.at[osl])


def _tc_b_body(x_ref, degp_ref, w_ref, y_ref, dis_ref):
    deg = jnp.sum(jnp.sum(degp_ref[...], axis=0), axis=1, keepdims=True) + 1.0
    disb = jnp.broadcast_to(lax.rsqrt(deg), (BLK, D))
    xw = jnp.dot(x_ref[...], w_ref[...], preferred_element_type=jnp.float32)
    y_ref[...] = xw * disb
    dis_ref[...] = disb


def _tc_d_body(p_ref, dis_ref, w_ref, b_ref, y2_ref):
    dis = dis_ref[...]
    h = jnp.maximum(p_ref[...] * dis + b_ref[...], 0.0)
    y2_ref[...] = jnp.dot(h, w_ref[...], preferred_element_type=jnp.float32) * dis


def _tc_f_body(p_ref, dis_ref, b_ref, o_ref):
    o_ref[...] = p_ref[...] * dis_ref[...] + b_ref[...]


_row_spec = pl.BlockSpec((BLK, D), lambda i: (i, 0))
_full_spec = pl.BlockSpec((D, D), lambda i: (0, 0))
_bias_spec = pl.BlockSpec((1, D), lambda i: (0, 0))
_rows_f32 = jax.ShapeDtypeStruct((N_PAD, D), jnp.float32)

_tc_b = pl.pallas_call(
    _tc_b_body,
    grid=(GRID,),
    in_specs=[_row_spec, pl.BlockSpec((NW, BLK, 8), lambda i: (0, i, 0)),
              _full_spec],
    out_specs=[_row_spec, _row_spec],
    out_shape=[_rows_f32, _rows_f32],
)

_tc_d = pl.pallas_call(
    _tc_d_body,
    grid=(GRID,),
    in_specs=[_row_spec, _row_spec, _full_spec, _bias_spec],
    out_specs=_row_spec,
    out_shape=_rows_f32,
)

_tc_f = pl.pallas_call(
    _tc_f_body,
    grid=(GRID,),
    in_specs=[_row_spec, _row_spec, _bias_spec],
    out_specs=_row_spec,
    out_shape=_rows_f32,
)


def kernel(x, edge_index, W1, b1, W2, b2):
    src = edge_index[0].astype(jnp.int32)
    dst = edge_index[1].astype(jnp.int32)

    pad = jnp.full((E_PAD - E,), N, dtype=jnp.int32)
    srcp = jnp.concatenate([src, pad]).reshape(NROW, CH)
    dstp = jnp.concatenate([dst, pad]).reshape(NROW, CH)

    x_pad = jnp.pad(x, ((0, N_PAD - N), (0, 0)))
    zeros_8 = jnp.zeros((N_PAD, 8), jnp.float32)
    dmy_s = jnp.full((PCH, CH), N, jnp.int32)
    dmy_d = jnp.full((PCH, CH), QN, jnp.int32)

    degp = _deg_kernel(dstp, zeros_8)
    y1, dis = _tc_b(x_pad, degp, W1)
    p1 = _edge_kernel(y1, srcp, dstp, dmy_s, dmy_d)
    y2 = _tc_d(p1, dis, W2, b1.reshape(1, D))
    p2 = _edge_kernel(y2, srcp, dstp, dmy_s, dmy_d)
    out = _tc_f(p2, dis, b2.reshape(1, D))
    return out[:N]


# speedup vs baseline: 49.4570x; 49.4570x over previous
import functools

import jax
import jax.numpy as jnp
from jax import lax
from jax.experimental import pallas as pl
from jax.experimental.pallas import tpu as pltpu
from jax.experimental.pallas import tpu_sc as plsc

N = 10000
E = 320000
D = 128

NC = 2
NS = 16
NW = NC * NS

N_PAD = 10240
HALF_N = N_PAD // NC
QN = N_PAD // 4
ACC_ROWS = QN + 8
QROWS_PT = QN // NS
CH = 128
NROW = 2560
E_PAD = NROW * CH
NCH_DEG = NROW // NW
NCH_EDGE = NROW // NS
NPASS = 2
PCH = NCH_EDGE // NPASS
GRP = 4
EGRP = 4
GEDGES = EGRP * CH
CAP = PCH * CH + GEDGES + 256

BLK = 1280
GRID = N_PAD // BLK

_mesh = plsc.VectorSubcoreMesh(core_axis_name="c", subcore_axis_name="s")
_sc_params = pltpu.CompilerParams(use_tc_tiling_on_sc=False,
                                  needs_layout_passes=False)


@functools.partial(
    pl.kernel,
    out_type=jax.ShapeDtypeStruct((NW, N_PAD, 8), jnp.float32),
    mesh=_mesh,
    scratch_types=[
        pltpu.VMEM((NCH_DEG, CH), jnp.int32),
        pltpu.VMEM((N_PAD, 8), jnp.float32),
    ],
    compiler_params=_sc_params,
)
def _deg_kernel(dst_hbm, zeros_hbm, out_hbm, dst_v, hist):
    c = lax.axis_index("c")
    t = lax.axis_index("s")
    w = t * NC + c

    pltpu.sync_copy(dst_hbm.at[pl.ds(w * NCH_DEG, NCH_DEG)], dst_v)
    pltpu.sync_copy(zeros_hbm, hist)

    lanes = lax.iota(jnp.int32, 16)
    cols = lanes & 7
    mlow = lanes < 8
    mhigh = lanes >= 8
    ones_v = jnp.ones((16,), jnp.float32)

    def body(r, carry):
        for q in range(CH // 16):
            d_ = dst_v[r, pl.ds(16 * q, 16)]
            plsc.addupdate_scatter(hist, [d_, cols], ones_v, mask=mlow)
            plsc.addupdate_scatter(hist, [d_, cols], ones_v, mask=mhigh)
        return carry

    lax.fori_loop(0, NCH_DEG, body, 0)

    pltpu.sync_copy(hist, out_hbm.at[w])


@functools.partial(
    pl.kernel,
    out_type=jax.ShapeDtypeStruct((N_PAD, D), jnp.float32),
    mesh=_mesh,
    scratch_types=[
        pltpu.VMEM((PCH, CH), jnp.int32),
        pltpu.VMEM((PCH, CH), jnp.int32),
        pltpu.VMEM((PCH, CH), jnp.int32),
        pltpu.VMEM((PCH, CH), jnp.int32),
        pltpu.VMEM((EGRP, CH, D), jnp.float32),
        pltpu.VMEM_SHARED((ACC_ROWS, D), jnp.float32),
        pltpu.SemaphoreType.DMA,
        pltpu.SemaphoreType.DMA,
    ],
    compiler_params=_sc_params,
)
def _edge_kernel(y_hbm, src_hbm, dst_hbm, dmy_s_hbm, dmy_d_hbm, out_hbm,
                 src_v, dst_v, csrc, cdst, rows_v, acc, sem, sem_s):
    c = lax.axis_index("c")
    t = lax.axis_index("s")
    asl = pl.ds(t * QROWS_PT, QROWS_PT)

    for phase in range(2):
        lo = (phase * NC + c) * QN
        osl = pl.ds(lo + t * QROWS_PT, QROWS_PT)

        pltpu.sync_copy(y_hbm.at[osl], acc.at[asl])
        plsc.subcore_barrier()

        for p in range(NPASS):
            row0 = t * NCH_EDGE + p * PCH
            pltpu.sync_copy(src_hbm.at[pl.ds(row0, PCH)], src_v)
            pltpu.sync_copy(dst_hbm.at[pl.ds(row0, PCH)], dst_v)
            pltpu.sync_copy(dmy_s_hbm, csrc)
            pltpu.sync_copy(dmy_d_hbm, cdst)

            def crow(r, cnt_vec):
                for q in range(CH // 16):
                    s = src_v[r, pl.ds(16 * q, 16)]
                    d_ = dst_v[r, pl.ds(16 * q, 16)]
                    m = (d_ >= lo) & (d_ < lo + QN)
                    dl = d_ - lo
                    m_i = jnp.where(m, 1, 0)
                    pos = plsc.cumsum(m_i) - m_i + cnt_vec
                    prow = lax.shift_right_logical(pos, 7)
                    pcol = lax.bitwise_and(pos, 127)
                    plsc.store_scatter(csrc, [prow, pcol], s, mask=m)
                    plsc.store_scatter(cdst, [prow, pcol], dl, mask=m)
                    cnt_vec = cnt_vec + plsc.all_reduce_population_count(m)
                return cnt_vec

            cnt_vec = lax.fori_loop(0, PCH, crow, jnp.zeros((16,), jnp.int32))
            cnt = jnp.sum(cnt_vec) // 16

            def sbody(g, carry):
                @pl.when(g * GEDGES < cnt)
                def _():
                    run_group(g)
                return carry

            def run_group(g):
                gathers = [
                    pltpu.async_copy(y_hbm.at[csrc.at[g * EGRP + k]],
                                     rows_v.at[k], sem)
                    for k in range(EGRP)
                ]
                for g_ in gathers:
                    g_.wait()
                scatters = [
                    pltpu.async_copy(rows_v.at[k],
                                     acc.at[cdst.at[g * EGRP + k]],
                                     sem_s, add=True)
                    for k in range(EGRP)
                ]
                for s_ in scatters:
                    s_.wait()

            lax.fori_loop(0, PCH // EGRP, sbody, 0)

        plsc.subcore_barrier()
        pltpu.sync_copy(acc.at[asl], out_hbm.at[osl])


def _tc_b_body(x_ref, degp_ref, w_ref, y_ref, dis_ref):
    deg = jnp.sum(jnp.sum(degp_ref[...], axis=0), axis=1, keepdims=True) + 1.0
    ridx = pl.program_id(0) * BLK + lax.broadcasted_iota(jnp.int32, (BLK, 1), 0)
    dis = jnp.where(ridx < N, lax.rsqrt(deg), 0.0)
    disb = jnp.broadcast_to(dis, (BLK, D))
    xw = jnp.dot(x_ref[...], w_ref[...], preferred_element_type=jnp.float32)
    y_ref[...] = xw * disb
    dis_ref[...] = disb


def _tc_d_body(p_ref, dis_ref, w_ref, b_ref, y2_ref):
    dis = dis_ref[...]
    h = jnp.maximum(p_ref[...] * dis + b_ref[...], 0.0)
    y2_ref[...] = jnp.dot(h, w_ref[...], preferred_element_type=jnp.float32) * dis


def _tc_f_body(p_ref, dis_ref, b_ref, o_ref):
    o_ref[...] = p_ref[...] * dis_ref[...] + b_ref[...]


_row_spec = pl.BlockSpec((BLK, D), lambda i: (i, 0))
_full_spec = pl.BlockSpec((D, D), lambda i: (0, 0))
_bias_spec = pl.BlockSpec((1, D), lambda i: (0, 0))
_rows_f32 = jax.ShapeDtypeStruct((N_PAD, D), jnp.float32)

_tc_b = pl.pallas_call(
    _tc_b_body,
    grid=(GRID,),
    in_specs=[_row_spec, pl.BlockSpec((NW, BLK, 8), lambda i: (0, i, 0)),
              _full_spec],
    out_specs=[_row_spec, _row_spec],
    out_shape=[_rows_f32, _rows_f32],
)

_tc_d = pl.pallas_call(
    _tc_d_body,
    grid=(GRID,),
    in_specs=[_row_spec, _row_spec, _full_spec, _bias_spec],
    out_specs=_row_spec,
    out_shape=_rows_f32,
)

_tc_f = pl.pallas_call(
    _tc_f_body,
    grid=(GRID,),
    in_specs=[_row_spec, _row_spec, _bias_spec],
    out_specs=_row_spec,
    out_shape=_rows_f32,
)


def kernel(x, edge_index, W1, b1, W2, b2):
    src = edge_index[0].astype(jnp.int32)
    dst = edge_index[1].astype(jnp.int32)

    pad = jnp.full((E_PAD - E,), N, dtype=jnp.int32)
    srcp = jnp.concatenate([src, pad]).reshape(NROW, CH)
    dstp = jnp.concatenate([dst, pad]).reshape(NROW, CH)

    x_pad = jnp.pad(x, ((0, N_PAD - N), (0, 0)))
    zeros_8 = jnp.zeros((N_PAD, 8), jnp.float32)
    dmy_s = (N + jnp.arange(PCH * CH, dtype=jnp.int32) % (N_PAD - N)).reshape(PCH, CH)
    dmy_d = (jnp.arange(PCH * CH, dtype=jnp.int32) % QN).reshape(PCH, CH)

    degp = _deg_kernel(dstp, zeros_8)
    y1, dis = _tc_b(x_pad, degp, W1)
    p1 = _edge_kernel(y1, srcp, dstp, dmy_s, dmy_d)
    y2 = _tc_d(p1, dis, W2, b1.reshape(1, D))
    p2 = _edge_kernel(y2, srcp, dstp, dmy_s, dmy_d)
    out = _tc_f(p2, dis, b2.reshape(1, D))
    return out[:N]


# speedup vs baseline: 112.0875x; 2.2664x over previous
import functools

import jax
import jax.numpy as jnp
from jax import lax
from jax.experimental import pallas as pl
from jax.experimental.pallas import tpu as pltpu
from jax.experimental.pallas import tpu_sc as plsc

N = 10000
E = 320000
D = 128

NC = 2
NS = 16
NW = NC * NS

N_PAD = 10240
HALF_N = N_PAD // NC
QN = N_PAD // 4
ACC_ROWS = QN + 8
QROWS_PT = QN // NS
CH = 128
NROW = 2560
E_PAD = NROW * CH
NCH_DEG = NROW // NW
NCH_EDGE = NROW // NS
NPASS = 2
PCH = NCH_EDGE // NPASS
GRP = 4
EGRP = 4
GEDGES = EGRP * CH
CAP = PCH * CH + GEDGES + 256

BLK = 1280
GRID = N_PAD // BLK

_mesh = plsc.VectorSubcoreMesh(core_axis_name="c", subcore_axis_name="s")
_sc_params = pltpu.CompilerParams(use_tc_tiling_on_sc=False,
                                  needs_layout_passes=False)


@functools.partial(
    pl.kernel,
    out_type=jax.ShapeDtypeStruct((NW, N_PAD, 8), jnp.float32),
    mesh=_mesh,
    scratch_types=[
        pltpu.VMEM((NCH_DEG, CH), jnp.int32),
        pltpu.VMEM((N_PAD, 8), jnp.float32),
    ],
    compiler_params=_sc_params,
)
def _deg_kernel(dst_hbm, zeros_hbm, out_hbm, dst_v, hist):
    c = lax.axis_index("c")
    t = lax.axis_index("s")
    w = t * NC + c

    pltpu.sync_copy(dst_hbm.at[pl.ds(w * NCH_DEG, NCH_DEG)], dst_v)
    pltpu.sync_copy(zeros_hbm, hist)

    lanes = lax.iota(jnp.int32, 16)
    cols = lanes & 7
    mlow = lanes < 8
    mhigh = lanes >= 8
    ones_v = jnp.ones((16,), jnp.float32)

    def body(r, carry):
        for q in range(CH // 16):
            d_ = dst_v[r, pl.ds(16 * q, 16)]
            plsc.addupdate_scatter(hist, [d_, cols], ones_v, mask=mlow)
            plsc.addupdate_scatter(hist, [d_, cols], ones_v, mask=mhigh)
        return carry

    lax.fori_loop(0, NCH_DEG, body, 0)

    pltpu.sync_copy(hist, out_hbm.at[w])


@functools.partial(
    pl.kernel,
    out_type=jax.ShapeDtypeStruct((N_PAD, D), jnp.float32),
    mesh=_mesh,
    scratch_types=[
        pltpu.VMEM((PCH, CH), jnp.int32),
        pltpu.VMEM((PCH, CH), jnp.int32),
        pltpu.VMEM((PCH, CH), jnp.int32),
        pltpu.VMEM((PCH, CH), jnp.int32),
        pltpu.VMEM((EGRP, CH, D), jnp.float32),
        pltpu.VMEM_SHARED((ACC_ROWS, D), jnp.float32),
        pltpu.SemaphoreType.DMA,
        pltpu.SemaphoreType.DMA,
    ],
    compiler_params=_sc_params,
)
def _edge_kernel(y_hbm, src_hbm, dst_hbm, dmy_s_hbm, dmy_d_hbm, out_hbm,
                 src_v, dst_v, csrc, cdst, rows_v, acc, sem, sem_s):
    c = lax.axis_index("c")
    t = lax.axis_index("s")
    asl = pl.ds(t * QROWS_PT, QROWS_PT)

    for phase in range(2):
        lo = (phase * NC + c) * QN
        osl = pl.ds(lo + t * QROWS_PT, QROWS_PT)

        pltpu.sync_copy(y_hbm.at[osl], acc.at[asl])
        plsc.subcore_barrier()

        for p in range(NPASS):
            row0 = t * NCH_EDGE + p * PCH
            pltpu.sync_copy(src_hbm.at[pl.ds(row0, PCH)], src_v)
            pltpu.sync_copy(dst_hbm.at[pl.ds(row0, PCH)], dst_v)
            pltpu.sync_copy(dmy_s_hbm, csrc)
            pltpu.sync_copy(dmy_d_hbm, cdst)

            def crow(r, cnt_vec):
                for q in range(CH // 16):
                    s = src_v[r, pl.ds(16 * q, 16)]
                    d_ = dst_v[r, pl.ds(16 * q, 16)]
                    m = (d_ >= lo) & (d_ < lo + QN)
                    dl = d_ - lo
                    m_i = jnp.where(m, 1, 0)
                    pos = plsc.cumsum(m_i) - m_i + cnt_vec
                    prow = lax.shift_right_logical(pos, 7)
                    pcol = lax.bitwise_and(pos, 127)
                    plsc.store_scatter(csrc, [prow, pcol], s, mask=m)
                    plsc.store_scatter(cdst, [prow, pcol], dl, mask=m)
                    cnt_vec = cnt_vec + plsc.all_reduce_population_count(m)
                return cnt_vec

            cnt_vec = lax.fori_loop(0, PCH, crow, jnp.zeros((16,), jnp.int32))
            cnt = jnp.sum(cnt_vec) // 16

            def sbody(g, carry):
                @pl.when(g * GEDGES < cnt)
                def _():
                    run_group(g)
                return carry

            def run_group(g):
                gathers = [
                    pltpu.async_copy(y_hbm.at[csrc.at[g * EGRP + k]],
                                     rows_v.at[k], sem)
                    for k in range(EGRP)
                ]
                for g_ in gathers:
                    g_.wait()
                scatters = [
                    pltpu.async_copy(rows_v.at[k],
                                     acc.at[cdst.at[g * EGRP + k]],
                                     sem_s, add=True)
                    for k in range(EGRP)
                ]
                for s_ in scatters:
                    s_.wait()

            lax.fori_loop(0, PCH // EGRP, sbody, 0)

        plsc.subcore_barrier()
        pltpu.sync_copy(acc.at[asl], out_hbm.at[osl])


def _tc_b_body(x_ref, degp_ref, w_ref, y_ref, dis_ref):
    deg = jnp.sum(jnp.sum(degp_ref[...], axis=0), axis=1, keepdims=True) + 1.0
    ridx = pl.program_id(0) * BLK + lax.broadcasted_iota(jnp.int32, (BLK, 1), 0)
    dis = jnp.where(ridx < N, lax.rsqrt(deg), 0.0)
    disb = jnp.broadcast_to(dis, (BLK, D))
    xw = jnp.dot(x_ref[...], w_ref[...], preferred_element_type=jnp.float32)
    y_ref[...] = xw * disb
    dis_ref[...] = disb


def _tc_d_body(p_ref, dis_ref, w_ref, b_ref, y2_ref):
    dis = dis_ref[...]
    h = jnp.maximum(p_ref[...] * dis + b_ref[...], 0.0)
    y2_ref[...] = jnp.dot(h, w_ref[...], preferred_element_type=jnp.float32) * dis


def _tc_f_body(p_ref, dis_ref, b_ref, o_ref):
    o_ref[...] = p_ref[...] * dis_ref[...] + b_ref[...]


_row_spec = pl.BlockSpec((BLK, D), lambda i: (i, 0))
_full_spec = pl.BlockSpec((D, D), lambda i: (0, 0))
_bias_spec = pl.BlockSpec((1, D), lambda i: (0, 0))
_rows_f32 = jax.ShapeDtypeStruct((N_PAD, D), jnp.float32)

_tc_b = pl.pallas_call(
    _tc_b_body,
    grid=(GRID,),
    in_specs=[_row_spec, pl.BlockSpec((NW, BLK, 8), lambda i: (0, i, 0)),
              _full_spec],
    out_specs=[_row_spec, _row_spec],
    out_shape=[_rows_f32, _rows_f32],
)

_tc_d = pl.pallas_call(
    _tc_d_body,
    grid=(GRID,),
    in_specs=[_row_spec, _row_spec, _full_spec, _bias_spec],
    out_specs=_row_spec,
    out_shape=_rows_f32,
)

_tc_f = pl.pallas_call(
    _tc_f_body,
    grid=(GRID,),
    in_specs=[_row_spec, _row_spec, _bias_spec],
    out_specs=_row_spec,
    out_shape=_rows_f32,
)


def kernel(x, edge_index, W1, b1, W2, b2):
    src = edge_index[0].astype(jnp.int32)
    dst = edge_index[1].astype(jnp.int32)

    npad = E_PAD - E
    pad_s = N + jnp.arange(npad, dtype=jnp.int32) % (N_PAD - N)
    pad_d = jnp.arange(npad, dtype=jnp.int32) % N_PAD
    srcp = jnp.concatenate([src, pad_s]).reshape(NROW, CH)
    dstp = jnp.concatenate([dst, pad_d]).reshape(NROW, CH)
    dstp_deg = jnp.concatenate(
        [dst, jnp.full((npad,), N, dtype=jnp.int32)]).reshape(NROW, CH)

    x_pad = jnp.pad(x, ((0, N_PAD - N), (0, 0)))
    zeros_8 = jnp.zeros((N_PAD, 8), jnp.float32)
    dmy_s = (N + jnp.arange(PCH * CH, dtype=jnp.int32) % (N_PAD - N)).reshape(PCH, CH)
    dmy_d = (jnp.arange(PCH * CH, dtype=jnp.int32) % QN).reshape(PCH, CH)

    degp = _deg_kernel(dstp_deg, zeros_8)
    y1, dis = _tc_b(x_pad, degp, W1)
    p1 = _edge_kernel(y1, srcp, dstp, dmy_s, dmy_d)
    y2 = _tc_d(p1, dis, W2, b1.reshape(1, D))
    p2 = _edge_kernel(y2, srcp, dstp, dmy_s, dmy_d)
    out = _tc_f(p2, dis, b2.reshape(1, D))
    return out[:N]
